# baseline (speedup 1.0000x reference)
import functools

import jax
import jax.numpy as jnp
from jax import lax
from jax.experimental import pallas as pl
from jax.experimental.pallas import tpu as pltpu
from jax.experimental.pallas import tpu_sc as plsc

_VOCAB = 1000000
_EMB = 64
_LABEL = 5
_B = 4096
_L = 200
_LH = _L // 2
_NC = 2
_NS = 16
_NW = _NC * _NS
_BPW = _B // _NW
_LANES = 16
_NV = _EMB // _LANES
_NBUF = 4
_W2 = 2 * _EMB
_VCHUNK = 2048
_NBLK = 123
_S4 = _NBLK * _VCHUNK
_BMAX = (_VOCAB - 1) // _VCHUNK


def _tc_transpose(a_ref, b_ref, c_ref, d_ref, e_ref, o_ref):
    eye = e_ref[...]
    half = jnp.int32(0x8000)

    def tp(r):
        t = lax.dot_general(
            r[...], eye, (((0,), (0,)), ((), ())),
            preferred_element_type=jnp.float32)
        bits = lax.bitcast_convert_type(t, jnp.int32)
        lo = lax.shift_right_logical(bits[:, 0:32] + half, 16)
        hi = jnp.bitwise_and(bits[:, 32:64] + half, jnp.int32(-65536))
        return jnp.bitwise_or(lo, hi)

    o_ref[...] = jnp.concatenate(
        [tp(a_ref), tp(b_ref), tp(c_ref), tp(d_ref)], axis=-1)


def _repack_table(table_t):
    eye = jnp.eye(_EMB, dtype=jnp.float32)
    return pl.pallas_call(
        _tc_transpose,
        grid=(_NBLK,),
        in_specs=[
            pl.BlockSpec((_EMB, _VCHUNK), lambda i: (0, i)),
            pl.BlockSpec((_EMB, _VCHUNK), lambda i: (0, i + _NBLK)),
            pl.BlockSpec((_EMB, _VCHUNK), lambda i: (0, i + 2 * _NBLK)),
            pl.BlockSpec((_EMB, _VCHUNK),
                         lambda i: (0, jnp.minimum(i + 3 * _NBLK, _BMAX))),
            pl.BlockSpec((_EMB, _EMB), lambda i: (0, 0)),
        ],
        out_specs=pl.BlockSpec((_VCHUNK, 2 * _EMB), lambda i: (i, 0)),
        out_shape=jax.ShapeDtypeStruct((_S4, 2 * _EMB), jnp.int32),
    )(table_t, table_t, table_t, table_t, eye)


def _sc_sum(rows3, table_lin):
    mesh = plsc.VectorSubcoreMesh(core_axis_name="c", subcore_axis_name="s")

    @functools.partial(
        pl.kernel,
        out_type=jax.ShapeDtypeStruct((_B, _EMB), jnp.float32),
        mesh=mesh,
        scratch_types=[
            pltpu.VMEM((_BPW, 2, _LH), jnp.int32),
            pltpu.VMEM((_NBUF, _L, _EMB // 2), jnp.int32),
            pltpu.VMEM((_BPW, _EMB), jnp.float32),
            pltpu.SemaphoreType.DMA,
            pltpu.SemaphoreType.DMA,
            pltpu.SemaphoreType.DMA,
            pltpu.SemaphoreType.DMA,
        ],
        compiler_params=pltpu.CompilerParams(use_tc_tiling_on_sc=False),
    )
    def k(r_hbm, table_hbm, out_hbm, idx_v, rows_v, out_v, s0, s1, s2, s3):
        sems = [s0, s1, s2, s3]
        wid = lax.axis_index("s") * _NC + lax.axis_index("c")
        base = wid * _BPW
        pltpu.sync_copy(r_hbm.at[pl.ds(base, _BPW)], idx_v)

        def issue(i, j):
            pltpu.async_copy(
                table_hbm.at[idx_v.at[i, 0]], rows_v.at[j, pl.ds(0, _LH)],
                sems[j])
            pltpu.async_copy(
                table_hbm.at[idx_v.at[i, 1]], rows_v.at[j, pl.ds(_LH, _LH)],
                sems[j])

        for j in range(_NBUF):
            issue(j, j)

        def group(g, _):
            for j in range(_NBUF):
                i = g * _NBUF + j
                pltpu.make_async_copy(
                    table_hbm.at[pl.ds(0, _L)], rows_v.at[j], sems[j]).wait()

                zero = jnp.zeros((_LANES,), jnp.float32)
                accs = [zero] * (2 * _NV)

                def red(lb, accs, j=j):
                    accs = list(accs)
                    for u in range(8):
                        l = lb * 8 + u
                        p = (u % 2) * _NV
                        for c in range(2):
                            x = rows_v[j, l, pl.ds(c * _LANES, _LANES)]
                            e = lax.bitcast_convert_type(
                                jnp.left_shift(x, 16), jnp.float32)
                            o = lax.bitcast_convert_type(
                                jnp.bitwise_and(x, jnp.int32(-65536)),
                                jnp.float32)
                            accs[p + 2 * c] = accs[p + 2 * c] + e
                            accs[p + 2 * c + 1] = accs[p + 2 * c + 1] + o
                    return tuple(accs)

                accs = lax.fori_loop(0, _L // 8, red, tuple(accs))
                for d in range(_NV):
                    out_v[i, pl.ds(d * _LANES, _LANES)] = (
                        accs[d] + accs[_NV + d])

                @pl.when(i + _NBUF < _BPW)
                def _(i=i, j=j):
                    issue(i + _NBUF, j)
            return 0

        lax.fori_loop(0, _BPW // _NBUF, group, 0)
        pltpu.sync_copy(out_v, out_hbm.at[pl.ds(base, _BPW)])

    return k(rows3, table_lin)


def _tc_head(x_ref, w_ref, b_ref, o_ref):
    x = x_ref[...] * (1.0 / _L)
    logits = lax.dot_general(
        x, w_ref[...], (((1,), (1,)), ((), ())),
        preferred_element_type=jnp.float32) + b_ref[...]
    m = jnp.max(logits, axis=1, keepdims=True)
    s = logits - m
    o_ref[...] = s - jnp.log(jnp.sum(jnp.exp(s), axis=1, keepdims=True))


def kernel(src, src_lengths, emb_table, W, b):
    del src_lengths
    src_i = src.astype(jnp.int32)
    q = src_i // _S4
    rows3 = (4 * (src_i - q * _S4) + q).reshape(_B, 2, _LH)
    table_lin = _repack_table(emb_table.T).reshape(4 * _S4, _EMB // 2)
    sums = _sc_sum(rows3, table_lin)
    perm = jnp.array(
        [g * 16 + k for g in (0, 2, 1, 3) for k in range(16)],
        dtype=jnp.int32)
    return pl.pallas_call(
        _tc_head,
        out_shape=jax.ShapeDtypeStruct((_B, _LABEL), jnp.float32),
    )(sums, W[:, perm], b.reshape(1, _LABEL))


# speedup vs baseline: 1.1607x; 1.1607x over previous
import functools

import jax
import jax.numpy as jnp
from jax import lax
from jax.experimental import pallas as pl
from jax.experimental.pallas import tpu as pltpu
from jax.experimental.pallas import tpu_sc as plsc

_VOCAB = 1000000
_EMB = 64
_LABEL = 5
_B = 4096
_L = 200
_LH = _L // 2
_NC = 2
_NS = 16
_NW = _NC * _NS
_BPW = _B // _NW
_LANES = 16
_NV = _EMB // _LANES
_NBUF = 4
_W2 = 2 * _EMB
_VCHUNK = 4096
_NBLK = 123
_S = _NBLK * _VCHUNK
_BMAX = (_VOCAB - 1) // _VCHUNK


def _tc_transpose(a_ref, b_ref, e_ref, o_ref):
    eye = e_ref[...]
    ta = lax.dot_general(
        a_ref[...], eye, (((0,), (0,)), ((), ())),
        preferred_element_type=jnp.float32)
    tb = lax.dot_general(
        b_ref[...], eye, (((0,), (0,)), ((), ())),
        preferred_element_type=jnp.float32)
    o_ref[...] = jnp.concatenate([ta, tb], axis=-1)


def _repack_table(table_t):
    eye = jnp.eye(_EMB, dtype=jnp.float32)
    return pl.pallas_call(
        _tc_transpose,
        grid=(_NBLK,),
        in_specs=[
            pl.BlockSpec((_EMB, _VCHUNK), lambda i: (0, i)),
            pl.BlockSpec((_EMB, _VCHUNK),
                         lambda i: (0, jnp.minimum(i + _NBLK, _BMAX))),
            pl.BlockSpec((_EMB, _EMB), lambda i: (0, 0)),
        ],
        out_specs=pl.BlockSpec((_VCHUNK, _W2), lambda i: (i, 0)),
        out_shape=jax.ShapeDtypeStruct((_S, _W2), jnp.float32),
    )(table_t, table_t, eye)


def _sc_sum(rows2, table_lin):
    mesh = plsc.VectorSubcoreMesh(core_axis_name="c", subcore_axis_name="s")

    @functools.partial(
        pl.kernel,
        out_type=jax.ShapeDtypeStruct((_B, _EMB), jnp.float32),
        mesh=mesh,
        scratch_types=[
            pltpu.VMEM((_BPW, _L), jnp.int32),
            pltpu.VMEM((_NBUF, _L, _EMB), jnp.float32),
            pltpu.VMEM((_BPW, _EMB), jnp.float32),
            pltpu.SemaphoreType.DMA,
            pltpu.SemaphoreType.DMA,
            pltpu.SemaphoreType.DMA,
            pltpu.SemaphoreType.DMA,
        ],
        compiler_params=pltpu.CompilerParams(use_tc_tiling_on_sc=False),
    )
    def k(r_hbm, table_hbm, out_hbm, idx_v, rows_v, out_v, s0, s1, s2, s3):
        sems = [s0, s1, s2, s3]
        wid = lax.axis_index("s") * _NC + lax.axis_index("c")
        base = wid * _BPW
        pltpu.sync_copy(r_hbm.at[pl.ds(base, _BPW)], idx_v)

        def issue(i, j):
            pltpu.async_copy(
                table_hbm.at[idx_v.at[i, pl.ds(0, 104)]], rows_v.at[j, pl.ds(0, 104)],
                sems[j])
            pltpu.async_copy(
                table_hbm.at[idx_v.at[i, pl.ds(104, 96)]], rows_v.at[j, pl.ds(104, 96)],
                sems[j])

        for j in range(_NBUF):
            issue(j, j)

        def group(g, _):
            for j in range(_NBUF):
                i = g * _NBUF + j
                pltpu.make_async_copy(
                    table_hbm.at[pl.ds(0, _L)], rows_v.at[j], sems[j]).wait()

                zero = jnp.zeros((_LANES,), jnp.float32)
                accs = [zero] * (2 * _NV)

                def red(lb, accs, j=j):
                    accs = list(accs)
                    for u in range(8):
                        l = lb * 8 + u
                        p = (u % 2) * _NV
                        for d in range(_NV):
                            accs[p + d] = accs[p + d] + rows_v[
                                j, l, pl.ds(d * _LANES, _LANES)]
                    return tuple(accs)

                accs = lax.fori_loop(0, _L // 8, red, tuple(accs))
                for d in range(_NV):
                    out_v[i, pl.ds(d * _LANES, _LANES)] = (
                        accs[d] + accs[_NV + d])

                @pl.when(i + _NBUF < _BPW)
                def _(i=i, j=j):
                    issue(i + _NBUF, j)
            return 0

        lax.fori_loop(0, _BPW // _NBUF, group, 0)
        pltpu.sync_copy(out_v, out_hbm.at[pl.ds(base, _BPW)])

    return k(rows2, table_lin)


def _tc_head(x_ref, w_ref, b_ref, o_ref):
    x = x_ref[...] * (1.0 / _L)
    logits = lax.dot_general(
        x, w_ref[...], (((1,), (1,)), ((), ())),
        preferred_element_type=jnp.float32) + b_ref[...]
    m = jnp.max(logits, axis=1, keepdims=True)
    s = logits - m
    o_ref[...] = s - jnp.log(jnp.sum(jnp.exp(s), axis=1, keepdims=True))


def kernel(src, src_lengths, emb_table, W, b):
    del src_lengths
    src_i = src.astype(jnp.int32)
    rows2 = jnp.where(src_i < _S, 2 * src_i, 2 * (src_i - _S) + 1)
    table_lin = _repack_table(emb_table.T).reshape(2 * _S, _EMB)
    sums = _sc_sum(rows2, table_lin)
    return pl.pallas_call(
        _tc_head,
        out_shape=jax.ShapeDtypeStruct((_B, _LABEL), jnp.float32),
    )(sums, W, b.reshape(1, _LABEL))
